# baseline (speedup 1.0000x reference)
import functools

import jax
import jax.numpy as jnp
from jax import lax
from jax.experimental import pallas as pl
from jax.experimental.pallas import tpu as pltpu
from jax.experimental.pallas import tpu_sc as plsc

_NBUF = 4


@functools.lru_cache(maxsize=None)
def _build(batch: int, seqlen: int, vocab: int, embed: int):
    info = plsc.get_sparse_core_info()
    nw = info.num_cores * info.num_subcores
    assert batch % nw == 0
    s_per_w = batch // nw
    assert s_per_w % _NBUF == 0
    nc = info.num_cores

    mesh = plsc.VectorSubcoreMesh(core_axis_name="c", subcore_axis_name="s")

    @functools.partial(
        pl.kernel,
        out_type=jax.ShapeDtypeStruct((batch, seqlen, embed), jnp.float32),
        mesh=mesh,
        scratch_types=[
            pltpu.VMEM((s_per_w, seqlen), jnp.int32),
            pltpu.VMEM((_NBUF, seqlen, embed), jnp.float32),
            pltpu.SemaphoreType.DMA((_NBUF,)),
            pltpu.SemaphoreType.DMA((_NBUF,)),
        ],
        compiler_params=pltpu.CompilerParams(use_tc_tiling_on_sc=False),
    )
    def gather_kernel(ids_hbm, table_hbm, out_hbm, idx_v, rows_v, gsem, ssem):
        wid = lax.axis_index("s") * nc + lax.axis_index("c")
        seq0 = wid * s_per_w

        pltpu.sync_copy(ids_hbm.at[pl.ds(seq0, s_per_w)], idx_v)

        def gather(buf, seq):
            return pltpu.make_async_copy(
                table_hbm.at[idx_v.at[seq]], rows_v.at[buf], gsem.at[buf])

        def scatter(buf, seq):
            return pltpu.make_async_copy(
                rows_v.at[buf], out_hbm.at[seq0 + seq], ssem.at[buf])

        for b in range(_NBUF):
            gather(b, b).start()

        @pl.loop(0, s_per_w, step=_NBUF)
        def _(j):
            for b in range(_NBUF):
                gather(b, j + b).wait()
                scatter(b, j + b).start()
            for b in range(_NBUF):
                scatter(b, j + b).wait()

                @pl.when(j + b + _NBUF < s_per_w)
                def _():
                    gather(b, j + b + _NBUF).start()

    return gather_kernel


def kernel(input_ids, embedding_table):
    b, l = input_ids.shape
    vocab, embed = embedding_table.shape
    fn = _build(b, l, vocab, embed)
    return fn(input_ids, embedding_table)


# speedup vs baseline: 1.0010x; 1.0010x over previous
import functools

import jax
import jax.numpy as jnp
from jax import lax
from jax.experimental import pallas as pl
from jax.experimental.pallas import tpu as pltpu
from jax.experimental.pallas import tpu_sc as plsc

_CHUNK = 512
_NBUF = 5


@functools.lru_cache(maxsize=None)
def _build(batch: int, seqlen: int, vocab: int, embed: int):
    info = plsc.get_sparse_core_info()
    nw = info.num_cores * info.num_subcores
    n_tokens = batch * seqlen
    assert n_tokens % (nw * _CHUNK) == 0
    per_w = n_tokens // nw
    nchunk = per_w // _CHUNK
    assert nchunk % _NBUF == 0
    nc = info.num_cores

    mesh = plsc.VectorSubcoreMesh(core_axis_name="c", subcore_axis_name="s")

    @functools.partial(
        pl.kernel,
        out_type=jax.ShapeDtypeStruct((n_tokens, embed), jnp.float32),
        mesh=mesh,
        scratch_types=[
            pltpu.VMEM((per_w,), jnp.int32),
            pltpu.VMEM((_NBUF, _CHUNK, embed), jnp.float32),
            pltpu.SemaphoreType.DMA((_NBUF,)),
            pltpu.SemaphoreType.DMA((_NBUF,)),
        ],
        compiler_params=pltpu.CompilerParams(use_tc_tiling_on_sc=False),
    )
    def gather_kernel(ids_hbm, table_hbm, out_hbm, idx_v, rows_v, gsem, ssem):
        wid = lax.axis_index("s") * nc + lax.axis_index("c")
        row0 = wid * per_w

        pltpu.sync_copy(ids_hbm.at[pl.ds(row0, per_w)], idx_v)

        def gather(buf, chunk):
            return pltpu.make_async_copy(
                table_hbm.at[idx_v.at[pl.ds(chunk * _CHUNK, _CHUNK)]],
                rows_v.at[buf], gsem.at[buf])

        def scatter(buf, chunk):
            return pltpu.make_async_copy(
                rows_v.at[buf],
                out_hbm.at[pl.ds(row0 + chunk * _CHUNK, _CHUNK)],
                ssem.at[buf])

        for b in range(_NBUF):
            gather(b, b).start()

        @pl.loop(0, nchunk, step=_NBUF)
        def _(j):
            for b in range(_NBUF):
                gather(b, j + b).wait()
                scatter(b, j + b).start()
            for b in range(_NBUF):
                scatter(b, j + b).wait()

                @pl.when(j + b + _NBUF < nchunk)
                def _():
                    gather(b, j + b + _NBUF).start()

    return gather_kernel


def kernel(input_ids, embedding_table):
    b, l = input_ids.shape
    vocab, embed = embedding_table.shape
    fn = _build(b, l, vocab, embed)
    out = fn(input_ids.reshape(b * l), embedding_table)
    return out.reshape(b, l, embed)
